# Initial kernel scaffold; baseline (speedup 1.0000x reference)
import jax
import jax.numpy as jnp
from jax.experimental import pallas as pl


def kernel(logits, labels):
    raise NotImplementedError("write your pallas kernel here")


# speedup vs baseline: 9.9616x; 9.9616x over previous
import functools

import jax
import jax.numpy as jnp
from jax import lax
from jax.experimental import pallas as pl
from jax.experimental.pallas import tpu as pltpu
from jax.experimental.pallas import tpu_sc as plsc

L = 16
B_IMG = 16
P = 512 * 512
NBINS = 8192
EMAX = 16.0
SCALE = NBINS / EMAX
CHUNK = 4096


def _body(logits_hbm, labels_hbm, out_hbm, lg_buf, lb_buf,
          hist_cnt, hist_pos, hist_sum, loss_buf):
    c = lax.axis_index("c")
    s = lax.axis_index("s")
    wid = s * 2 + c

    @pl.when(wid < B_IMG)
    def _():
        img = wid
        zeros = jnp.zeros((L,), jnp.float32)
        ones = jnp.ones((L,), jnp.float32)

        def zero_bins(i, _):
            hist_cnt[pl.ds(i * L, L)] = zeros
            hist_pos[pl.ds(i * L, L)] = zeros
            hist_sum[pl.ds(i * L, L)] = zeros
            return 0

        lax.fori_loop(0, NBINS // L, zero_bins, 0)

        def do_chunk(ch, g_acc):
            pltpu.sync_copy(logits_hbm.at[img, pl.ds(ch * CHUNK, CHUNK)], lg_buf)
            pltpu.sync_copy(labels_hbm.at[img, pl.ds(ch * CHUNK, CHUNK)], lb_buf)

            def do_vreg(k, g):
                lg = lg_buf[pl.ds(k * L, L)]
                lf = lb_buf[pl.ds(k * L, L)].astype(jnp.float32)
                e = (1.0 + lg) - 2.0 * (lg * lf)
                m = e > 0.0
                idx = jnp.clip((e * SCALE).astype(jnp.int32), 0, NBINS - 1)
                plsc.addupdate_scatter(hist_cnt, [idx], ones, mask=m)
                plsc.addupdate_scatter(hist_pos, [idx], lf, mask=m)
                plsc.addupdate_scatter(hist_sum, [idx], e, mask=m)
                return g + lf

            return lax.fori_loop(0, CHUNK // L, do_vreg, g_acc)

        g_vec = lax.fori_loop(0, P // CHUNK, do_chunk, zeros)
        g_tot = jnp.sum(g_vec)

        def scan_step(v, carry):
            n_hi, c_hi, loss_vec = carry
            base = NBINS - (v + 1) * L
            x = hist_cnt[pl.ds(base, L)]
            y = hist_pos[pl.ds(base, L)]
            z = hist_sum[pl.ds(base, L)]
            sx = plsc.cumsum(x)
            sy = plsc.cumsum(y)
            tx = jnp.sum(x)
            ty = jnp.sum(y)
            n0 = n_hi + (tx - sx)
            c0 = c_hi + (ty - sy)
            n1 = n0 + x
            c1 = c0 + y
            j0 = jnp.where(n0 < 0.5, 0.0,
                           1.0 - (g_tot - c0) / jnp.maximum(g_tot + n0 - c0, 1.0))
            j1 = jnp.where(n1 < 0.5, 0.0,
                           1.0 - (g_tot - c1) / jnp.maximum(g_tot + n1 - c1, 1.0))
            mean_e = z / jnp.maximum(x, 1.0)
            loss_vec = loss_vec + mean_e * (j1 - j0)
            return (n_hi + tx, c_hi + ty, loss_vec)

        _, _, loss_vec = lax.fori_loop(
            0, NBINS // L, scan_step,
            (jnp.float32(0.0), jnp.float32(0.0), zeros))
        loss_buf[...] = jnp.full((L,), jnp.sum(loss_vec), jnp.float32)
        pltpu.sync_copy(loss_buf, out_hbm.at[img])


@jax.jit
def kernel(logits, labels):
    logits2 = logits.reshape(B_IMG, P)
    labels2 = labels.reshape(B_IMG, P)
    mesh = plsc.VectorSubcoreMesh(core_axis_name="c", subcore_axis_name="s")
    out = pl.kernel(
        _body,
        out_type=jax.ShapeDtypeStruct((B_IMG, L), jnp.float32),
        mesh=mesh,
        scratch_types=[
            pltpu.VMEM((CHUNK,), jnp.float32),
            pltpu.VMEM((CHUNK,), jnp.int32),
            pltpu.VMEM((NBINS,), jnp.float32),
            pltpu.VMEM((NBINS,), jnp.float32),
            pltpu.VMEM((NBINS,), jnp.float32),
            pltpu.VMEM((L,), jnp.float32),
        ],
        compiler_params=pltpu.CompilerParams(needs_layout_passes=False),
    )(logits2, labels2)
    return jnp.mean(out[:, 0])


# speedup vs baseline: 11.3053x; 1.1349x over previous
import functools

import jax
import jax.numpy as jnp
from jax import lax
from jax.experimental import pallas as pl
from jax.experimental.pallas import tpu as pltpu
from jax.experimental.pallas import tpu_sc as plsc

L = 16
B_IMG = 16
P = 512 * 512
NBINS = 8192
EMAX = 16.0
SCALE = NBINS / EMAX
CHUNK = 16384


def _body(logits_hbm, labels_hbm, out_hbm, lg_buf, lb_buf,
          hist_cnt, hist_pos, hist_sum, loss_buf):
    c = lax.axis_index("c")
    s = lax.axis_index("s")
    wid = s * 2 + c

    @pl.when(wid < B_IMG)
    def _():
        img = wid
        zeros = jnp.zeros((L,), jnp.float32)
        ones = jnp.ones((L,), jnp.float32)

        def zero_bins(i, _):
            hist_cnt[pl.ds(i * L, L)] = zeros
            hist_pos[pl.ds(i * L, L)] = zeros
            hist_sum[pl.ds(i * L, L)] = zeros
            return 0

        lax.fori_loop(0, NBINS // L, zero_bins, 0, unroll=4)

        def do_chunk(ch, g_acc):
            base = img * P + ch * CHUNK
            pltpu.sync_copy(logits_hbm.at[pl.ds(base, CHUNK)], lg_buf)
            pltpu.sync_copy(labels_hbm.at[pl.ds(base, CHUNK)], lb_buf)

            def do_vreg(k, g):
                lg = lg_buf[pl.ds(k * L, L)]
                lf = lb_buf[pl.ds(k * L, L)].astype(jnp.float32)
                e = (1.0 + lg) - 2.0 * (lg * lf)
                m = e > 0.0
                idx = jnp.clip((e * SCALE).astype(jnp.int32), 0, NBINS - 1)
                plsc.addupdate_scatter(hist_cnt, [idx], ones, mask=m)
                plsc.addupdate_scatter(hist_pos, [idx], lf, mask=m)
                plsc.addupdate_scatter(hist_sum, [idx], e, mask=m)
                return g + lf

            return lax.fori_loop(0, CHUNK // L, do_vreg, g_acc, unroll=4)

        g_vec = lax.fori_loop(0, P // CHUNK, do_chunk, zeros)
        g_tot = jnp.sum(g_vec)

        def scan_step(v, carry):
            n_hi, c_hi, loss_vec = carry
            base = NBINS - (v + 1) * L
            x = hist_cnt[pl.ds(base, L)]
            y = hist_pos[pl.ds(base, L)]
            z = hist_sum[pl.ds(base, L)]
            sx = plsc.cumsum(x)
            sy = plsc.cumsum(y)
            tx = jnp.sum(x)
            ty = jnp.sum(y)
            n0 = n_hi + (tx - sx)
            c0 = c_hi + (ty - sy)
            n1 = n0 + x
            c1 = c0 + y
            j0 = jnp.where(n0 < 0.5, 0.0,
                           1.0 - (g_tot - c0) / jnp.maximum(g_tot + n0 - c0, 1.0))
            j1 = jnp.where(n1 < 0.5, 0.0,
                           1.0 - (g_tot - c1) / jnp.maximum(g_tot + n1 - c1, 1.0))
            mean_e = z / jnp.maximum(x, 1.0)
            loss_vec = loss_vec + mean_e * (j1 - j0)
            return (n_hi + tx, c_hi + ty, loss_vec)

        _, _, loss_vec = lax.fori_loop(
            0, NBINS // L, scan_step,
            (jnp.float32(0.0), jnp.float32(0.0), zeros), unroll=2)
        loss_buf[...] = jnp.full((L,), jnp.sum(loss_vec), jnp.float32)
        pltpu.sync_copy(loss_buf, out_hbm.at[img])


@jax.jit
def kernel(logits, labels):
    logits1 = logits.reshape(-1)
    labels1 = labels.reshape(-1)
    mesh = plsc.VectorSubcoreMesh(core_axis_name="c", subcore_axis_name="s")
    out = pl.kernel(
        _body,
        out_type=jax.ShapeDtypeStruct((B_IMG, L), jnp.float32),
        mesh=mesh,
        scratch_types=[
            pltpu.VMEM((CHUNK,), jnp.float32),
            pltpu.VMEM((CHUNK,), jnp.int32),
            pltpu.VMEM((NBINS,), jnp.float32),
            pltpu.VMEM((NBINS,), jnp.float32),
            pltpu.VMEM((NBINS,), jnp.float32),
            pltpu.VMEM((L,), jnp.float32),
        ],
        compiler_params=pltpu.CompilerParams(needs_layout_passes=False),
    )(logits1, labels1)
    return jnp.mean(out[:, 0])


# speedup vs baseline: 18.7685x; 1.6602x over previous
import jax
import jax.numpy as jnp
from jax import lax
from jax.experimental import pallas as pl
from jax.experimental.pallas import tpu as pltpu
from jax.experimental.pallas import tpu_sc as plsc

L = 16
B_IMG = 16
P = 512 * 512
HALF = P // 2
NBINS = 8192
EMAX = 16.0
SCALE = NBINS / EMAX
CHUNK = 16384
ROW = 3 * NBINS + L


def _body(logits_hbm, labels_hbm, out_hbm, xh_hbm, lg_buf, lb_buf,
          hist_cnt, hist_pos, hist_sum, tmp_buf, loss_buf):
    c = lax.axis_index("c")
    s = lax.axis_index("s")
    img = c * 8 + s // 2
    half = s % 2
    r = c * 16 + s

    zeros = jnp.zeros((L,), jnp.float32)
    ones = jnp.ones((L,), jnp.float32)

    def zero_bins(i, _):
        hist_cnt[pl.ds(i * L, L)] = zeros
        hist_pos[pl.ds(i * L, L)] = zeros
        hist_sum[pl.ds(i * L, L)] = zeros
        return 0

    lax.fori_loop(0, NBINS // L, zero_bins, 0, unroll=4)

    def do_chunk(ch, g_acc):
        base = img * P + half * HALF + ch * CHUNK
        pltpu.sync_copy(logits_hbm.at[pl.ds(base, CHUNK)], lg_buf)
        pltpu.sync_copy(labels_hbm.at[pl.ds(base, CHUNK)], lb_buf)

        def do_vreg(k, g):
            lg = lg_buf[pl.ds(k * L, L)]
            lf = lb_buf[pl.ds(k * L, L)].astype(jnp.float32)
            e = (1.0 + lg) - 2.0 * (lg * lf)
            m = e > 0.0
            idx = jnp.clip((e * SCALE).astype(jnp.int32), 0, NBINS - 1)
            plsc.addupdate_scatter(hist_cnt, [idx], ones, mask=m)
            plsc.addupdate_scatter(hist_pos, [idx], lf, mask=m)
            plsc.addupdate_scatter(hist_sum, [idx], e, mask=m)
            return g + lf

        return lax.fori_loop(0, CHUNK // L, do_vreg, g_acc, unroll=4)

    g_vec = lax.fori_loop(0, HALF // CHUNK, do_chunk, zeros)

    @pl.when(half == 1)
    def _():
        pltpu.sync_copy(hist_cnt, xh_hbm.at[r, pl.ds(0, NBINS)])
        pltpu.sync_copy(hist_pos, xh_hbm.at[r, pl.ds(NBINS, NBINS)])
        pltpu.sync_copy(hist_sum, xh_hbm.at[r, pl.ds(2 * NBINS, NBINS)])
        loss_buf[...] = g_vec
        pltpu.sync_copy(loss_buf, xh_hbm.at[r, pl.ds(3 * NBINS, L)])

    plsc.subcore_barrier()

    @pl.when(half == 0)
    def _():
        def merge(hist, off):
            pltpu.sync_copy(xh_hbm.at[r + 1, pl.ds(off, NBINS)], tmp_buf)

            def add_vreg(i, _):
                hist[pl.ds(i * L, L)] = hist[pl.ds(i * L, L)] + tmp_buf[pl.ds(i * L, L)]
                return 0

            lax.fori_loop(0, NBINS // L, add_vreg, 0, unroll=4)

        merge(hist_cnt, 0)
        merge(hist_pos, NBINS)
        merge(hist_sum, 2 * NBINS)
        pltpu.sync_copy(xh_hbm.at[r + 1, pl.ds(3 * NBINS, L)], loss_buf)
        g_tot = jnp.sum(g_vec) + jnp.sum(loss_buf[...])

        def scan_step(v, carry):
            n_hi, c_hi, loss_vec = carry
            base = NBINS - (v + 1) * L
            x = hist_cnt[pl.ds(base, L)]
            y = hist_pos[pl.ds(base, L)]
            z = hist_sum[pl.ds(base, L)]
            sx = plsc.cumsum(x)
            sy = plsc.cumsum(y)
            tx = jnp.sum(x)
            ty = jnp.sum(y)
            n0 = n_hi + (tx - sx)
            c0 = c_hi + (ty - sy)
            n1 = n0 + x
            c1 = c0 + y
            j0 = jnp.where(n0 < 0.5, 0.0,
                           1.0 - (g_tot - c0) / jnp.maximum(g_tot + n0 - c0, 1.0))
            j1 = jnp.where(n1 < 0.5, 0.0,
                           1.0 - (g_tot - c1) / jnp.maximum(g_tot + n1 - c1, 1.0))
            mean_e = z / jnp.maximum(x, 1.0)
            loss_vec = loss_vec + mean_e * (j1 - j0)
            return (n_hi + tx, c_hi + ty, loss_vec)

        _, _, loss_vec = lax.fori_loop(
            0, NBINS // L, scan_step,
            (jnp.float32(0.0), jnp.float32(0.0), zeros), unroll=2)
        loss_buf[...] = jnp.full((L,), jnp.sum(loss_vec), jnp.float32)
        pltpu.sync_copy(loss_buf, out_hbm.at[img])


def _raw(logits1, labels1):
    mesh = plsc.VectorSubcoreMesh(core_axis_name="c", subcore_axis_name="s")
    out, _ = pl.kernel(
        _body,
        out_type=(jax.ShapeDtypeStruct((B_IMG, L), jnp.float32),
                  jax.ShapeDtypeStruct((32, ROW), jnp.float32)),
        mesh=mesh,
        scratch_types=[
            pltpu.VMEM((CHUNK,), jnp.float32),
            pltpu.VMEM((CHUNK,), jnp.int32),
            pltpu.VMEM((NBINS,), jnp.float32),
            pltpu.VMEM((NBINS,), jnp.float32),
            pltpu.VMEM((NBINS,), jnp.float32),
            pltpu.VMEM((NBINS,), jnp.float32),
            pltpu.VMEM((L,), jnp.float32),
        ],
        compiler_params=pltpu.CompilerParams(needs_layout_passes=False),
    )(logits1, labels1)
    return out


@jax.jit
def kernel(logits, labels):
    out = _raw(logits.reshape(-1), labels.reshape(-1))
    return jnp.mean(out[:, 0])


# speedup vs baseline: 21.4599x; 1.1434x over previous
import jax
import jax.numpy as jnp
from jax import lax
from jax.experimental import pallas as pl
from jax.experimental.pallas import tpu as pltpu
from jax.experimental.pallas import tpu_sc as plsc

L = 16
B_IMG = 16
P = 512 * 512
HALF = P // 2
NBINS = 4096
EMAX = 16.0
SCALE = NBINS / EMAX
CHUNK = 16384
ROW = 3 * NBINS + L


def _body(logits_hbm, labels_hbm, out_hbm, xh_hbm, lg_buf, lb_buf,
          lg_buf2, lb_buf2, hist_cnt, hist_pos, hist_sum, tmp_buf, loss_buf,
          sem_lg, sem_lb, sem_lg2, sem_lb2):
    c = lax.axis_index("c")
    s = lax.axis_index("s")
    img = c * 8 + s // 2
    half = s % 2
    r = c * 16 + s

    zeros = jnp.zeros((L,), jnp.float32)
    ones = jnp.ones((L,), jnp.float32)

    def zero_bins(i, _):
        hist_cnt[pl.ds(i * L, L)] = zeros
        hist_pos[pl.ds(i * L, L)] = zeros
        hist_sum[pl.ds(i * L, L)] = zeros
        return 0

    lax.fori_loop(0, NBINS // L, zero_bins, 0, unroll=4)

    base0 = img * P + half * HALF
    n_chunks = HALF // CHUNK
    bufs = ((lg_buf, lb_buf, sem_lg, sem_lb),
            (lg_buf2, lb_buf2, sem_lg2, sem_lb2))

    def start(ch, b):
        lg, lb, slg, slb = bufs[b]
        base = base0 + ch * CHUNK
        dl = pltpu.async_copy(logits_hbm.at[pl.ds(base, CHUNK)], lg, slg)
        db = pltpu.async_copy(labels_hbm.at[pl.ds(base, CHUNK)], lb, slb)
        return dl, db

    def consume(b, gs):
        lg_b, lb_b, _, _ = bufs[b]

        def one(k):
            lg = lg_b[pl.ds(k * L, L)]
            lf = lb_b[pl.ds(k * L, L)].astype(jnp.float32)
            e = (1.0 + lg) - 2.0 * (lg * lf)
            m = e > 0.0
            idx = jnp.minimum((e * SCALE).astype(jnp.int32), NBINS - 1)
            plsc.addupdate_scatter(hist_cnt, [idx], ones, mask=m)
            plsc.addupdate_scatter(hist_pos, [idx], ones, mask=m & (lf > 0.5))
            plsc.addupdate_scatter(hist_sum, [idx], e, mask=m)
            return lf

        def do_quad(q, gs):
            g0, g1, g2, g3 = gs
            k = q * 4
            return (g0 + one(k), g1 + one(k + 1), g2 + one(k + 2), g3 + one(k + 3))

        return lax.fori_loop(0, CHUNK // L // 4, do_quad, gs)

    gs = (zeros, zeros, zeros, zeros)
    pending = start(0, 0)
    for ch in range(n_chunks):
        b = ch % 2
        nxt = start(ch + 1, 1 - b) if ch + 1 < n_chunks else None
        pending[0].wait()
        pending[1].wait()
        gs = consume(b, gs)
        pending = nxt
    g0, g1, g2, g3 = gs
    g_vec = (g0 + g1) + (g2 + g3)

    @pl.when(half == 1)
    def _():
        pltpu.sync_copy(hist_cnt, xh_hbm.at[r, pl.ds(0, NBINS)])
        pltpu.sync_copy(hist_pos, xh_hbm.at[r, pl.ds(NBINS, NBINS)])
        pltpu.sync_copy(hist_sum, xh_hbm.at[r, pl.ds(2 * NBINS, NBINS)])
        loss_buf[...] = g_vec
        pltpu.sync_copy(loss_buf, xh_hbm.at[r, pl.ds(3 * NBINS, L)])

    plsc.subcore_barrier()

    @pl.when(half == 0)
    def _():
        def merge(hist, off):
            pltpu.sync_copy(xh_hbm.at[r + 1, pl.ds(off, NBINS)], tmp_buf)

            def add_vreg(i, _):
                hist[pl.ds(i * L, L)] = hist[pl.ds(i * L, L)] + tmp_buf[pl.ds(i * L, L)]
                return 0

            lax.fori_loop(0, NBINS // L, add_vreg, 0, unroll=4)

        merge(hist_cnt, 0)
        merge(hist_pos, NBINS)
        merge(hist_sum, 2 * NBINS)
        pltpu.sync_copy(xh_hbm.at[r + 1, pl.ds(3 * NBINS, L)], loss_buf)
        g_tot = jnp.sum(g_vec) + jnp.sum(loss_buf[...])

        def scan_step(v, carry):
            n_hi, c_hi, loss_vec = carry
            base = NBINS - (v + 1) * L
            x = hist_cnt[pl.ds(base, L)]
            y = hist_pos[pl.ds(base, L)]
            z = hist_sum[pl.ds(base, L)]
            sx = plsc.cumsum(x)
            sy = plsc.cumsum(y)
            tx = jnp.sum(x)
            ty = jnp.sum(y)
            n0 = n_hi + (tx - sx)
            c0 = c_hi + (ty - sy)
            n1 = n0 + x
            c1 = c0 + y
            j0 = jnp.where(n0 < 0.5, 0.0,
                           1.0 - (g_tot - c0) / jnp.maximum(g_tot + n0 - c0, 1.0))
            j1 = jnp.where(n1 < 0.5, 0.0,
                           1.0 - (g_tot - c1) / jnp.maximum(g_tot + n1 - c1, 1.0))
            mean_e = z / jnp.maximum(x, 1.0)
            loss_vec = loss_vec + mean_e * (j1 - j0)
            return (n_hi + tx, c_hi + ty, loss_vec)

        _, _, loss_vec = lax.fori_loop(
            0, NBINS // L, scan_step,
            (jnp.float32(0.0), jnp.float32(0.0), zeros), unroll=2)
        loss_buf[...] = jnp.full((L,), jnp.sum(loss_vec), jnp.float32)
        pltpu.sync_copy(loss_buf, out_hbm.at[img])


def _raw(logits1, labels1):
    mesh = plsc.VectorSubcoreMesh(core_axis_name="c", subcore_axis_name="s")
    out, _ = pl.kernel(
        _body,
        out_type=(jax.ShapeDtypeStruct((B_IMG, L), jnp.float32),
                  jax.ShapeDtypeStruct((32, ROW), jnp.float32)),
        mesh=mesh,
        scratch_types=[
            pltpu.VMEM((CHUNK,), jnp.float32),
            pltpu.VMEM((CHUNK,), jnp.int32),
            pltpu.VMEM((CHUNK,), jnp.float32),
            pltpu.VMEM((CHUNK,), jnp.int32),
            pltpu.VMEM((NBINS,), jnp.float32),
            pltpu.VMEM((NBINS,), jnp.float32),
            pltpu.VMEM((NBINS,), jnp.float32),
            pltpu.VMEM((NBINS,), jnp.float32),
            pltpu.VMEM((L,), jnp.float32),
            pltpu.SemaphoreType.DMA,
            pltpu.SemaphoreType.DMA,
            pltpu.SemaphoreType.DMA,
            pltpu.SemaphoreType.DMA,
        ],
        compiler_params=pltpu.CompilerParams(needs_layout_passes=False),
    )(logits1, labels1)
    return out


@jax.jit
def kernel(logits, labels):
    out = _raw(logits.reshape(-1), labels.reshape(-1))
    return jnp.mean(out[:, 0])
